# baseline (speedup 1.0000x reference)
import functools
import jax
import jax.numpy as jnp
from jax import lax
from jax.experimental import pallas as pl
from jax.experimental.pallas import tpu as pltpu
from jax.experimental.pallas import tpu_sc as plsc

_SEQ = 2048
_D = 256
_NC = 2
_NS = 16
_LANES = 16


def kernel(z, pos_embed):
    batch = z.shape[1]
    nw = _NC * _NS
    rows_per_w = _SEQ // nw
    nchunks = _D // _LANES

    mesh = plsc.VectorSubcoreMesh(core_axis_name="c", subcore_axis_name="s")

    @functools.partial(
        pl.kernel,
        mesh=mesh,
        out_type=jax.ShapeDtypeStruct((_SEQ * batch, _D), jnp.float32),
        scratch_types=[
            pltpu.VMEM((rows_per_w, _D), jnp.float32),
            pltpu.VMEM((batch, _D), jnp.float32),
            pltpu.VMEM((batch, _D), jnp.float32),
            pltpu.SemaphoreType.DMA,
            pltpu.SemaphoreType.DMA,
        ],
    )
    def sc_fill(pe_hbm, out_hbm, tab_v, blk0_v, blk1_v, sem0, sem1):
        wid = lax.axis_index("s") * _NC + lax.axis_index("c")
        base = wid * rows_per_w
        pltpu.sync_copy(pe_hbm.at[pl.ds(base, rows_per_w)], tab_v)
        bufs = (blk0_v, blk1_v)
        sems = (sem0, sem1)

        def build(i, blk_v):
            vs = [tab_v[i, pl.ds(c * _LANES, _LANES)] for c in range(nchunks)]

            def j_body(jj, c2):
                for u in range(16):
                    j = jj * 16 + u
                    for c in range(nchunks):
                        blk_v[j, pl.ds(c * _LANES, _LANES)] = vs[c]
                return c2

            lax.fori_loop(0, batch // 16, j_body, 0)

        def start(i, b):
            pltpu.async_copy(
                bufs[b], out_hbm.at[pl.ds((base + i) * batch, batch)], sems[b]
            )

        def drain(i, b):
            pltpu.make_async_copy(
                bufs[b], out_hbm.at[pl.ds((base + i) * batch, batch)], sems[b]
            ).wait()

        for b in range(2):
            build(b, bufs[b])
            start(b, b)

        def t_body(t, carry):
            for b in range(2):
                i = t * 2 + b
                drain(i, b)
                build(i, bufs[b])
                start(i, b)
            return carry

        lax.fori_loop(1, rows_per_w // 2, t_body, 0)
        for b in range(2):
            drain(rows_per_w - 2 + b, b)

    out = sc_fill(pos_embed)
    return out.reshape(_SEQ, batch, _D)


# speedup vs baseline: 1.2488x; 1.2488x over previous
import functools
import jax
import jax.numpy as jnp
from jax import lax
from jax.experimental import pallas as pl
from jax.experimental.pallas import tpu as pltpu
from jax.experimental.pallas import tpu_sc as plsc

_SEQ = 2048
_D = 256
_NC = 2
_NS = 16
_LANES = 16


def kernel(z, pos_embed):
    batch = z.shape[1]
    nw = _NC * _NS
    rows_per_w = _SEQ // nw
    nchunks = _D // _LANES

    mesh = plsc.VectorSubcoreMesh(core_axis_name="c", subcore_axis_name="s")

    @functools.partial(
        pl.kernel,
        mesh=mesh,
        out_type=jax.ShapeDtypeStruct((_SEQ * batch, _D), jnp.float32),
        scratch_types=[
            pltpu.VMEM((rows_per_w, _D), jnp.float32),
            pltpu.VMEM((batch, _D), jnp.float32),
            pltpu.VMEM((batch, _D), jnp.float32),
            pltpu.SemaphoreType.DMA,
            pltpu.SemaphoreType.DMA,
        ],
    )
    def sc_fill(pe_hbm, out_hbm, tab_v, blk0_v, blk1_v, sem0, sem1):
        wid = lax.axis_index("s") * _NC + lax.axis_index("c")
        base = wid * rows_per_w
        pltpu.sync_copy(pe_hbm.at[pl.ds(base, rows_per_w)], tab_v)
        bufs = (blk0_v, blk1_v)
        sems = (sem0, sem1)

        def build(i, blk_v):
            vs = [tab_v[i, pl.ds(c * _LANES, _LANES)] for c in range(nchunks)]

            def j_body(jj, c2):
                for u in range(8):
                    j = jj * 8 + u
                    for c in range(nchunks):
                        blk_v[j, pl.ds(c * _LANES, _LANES)] = vs[c]
                return c2

            lax.fori_loop(0, batch // 8, j_body, 0)

        half = batch // 2

        def start(i, b):
            pltpu.async_copy(
                bufs[b].at[pl.ds(0, half)],
                out_hbm.at[pl.ds((base + i) * batch, half)],
                sems[b],
            )
            pltpu.async_copy(
                bufs[b].at[pl.ds(half, half)],
                out_hbm.at[pl.ds((base + i) * batch + half, half)],
                sems[b],
            )

        def drain(i, b):
            for h in range(2):
                pltpu.make_async_copy(
                    bufs[b].at[pl.ds(h * half, half)],
                    out_hbm.at[pl.ds((base + i) * batch + h * half, half)],
                    sems[b],
                ).wait()

        for b in range(2):
            build(b, bufs[b])
            start(b, b)

        def t_body(t, carry):
            for b in range(2):
                i = t * 2 + b
                drain(i, b)
                build(i, bufs[b])
                start(i, b)
            return carry

        lax.fori_loop(1, rows_per_w // 2, t_body, 0)
        for b in range(2):
            drain(rows_per_w - 2 + b, b)

    out = sc_fill(pos_embed)
    return out.reshape(_SEQ, batch, _D)


# speedup vs baseline: 1.2626x; 1.0111x over previous
import functools
import jax
import jax.numpy as jnp
from jax import lax
from jax.experimental import pallas as pl
from jax.experimental.pallas import tpu as pltpu
from jax.experimental.pallas import tpu_sc as plsc

_SEQ = 2048
_D = 256
_NC = 2
_NS = 16
_LANES = 16
_S_BLK = 64


def _sc_gather(pos_embed):
    nw = _NC * _NS
    rows_per_w = _SEQ // nw

    mesh = plsc.VectorSubcoreMesh(core_axis_name="c", subcore_axis_name="s")

    @functools.partial(
        pl.kernel,
        mesh=mesh,
        out_type=jax.ShapeDtypeStruct((_SEQ, _D), jnp.float32),
        scratch_types=[
            pltpu.VMEM((rows_per_w,), jnp.int32),
            pltpu.VMEM((rows_per_w, _D), jnp.float32),
            pltpu.SemaphoreType.DMA,
        ],
    )
    def gather_k(table_hbm, out_hbm, idx_v, rows_v, sem):
        wid = lax.axis_index("s") * _NC + lax.axis_index("c")
        base = wid * rows_per_w
        for c in range(rows_per_w // _LANES):
            idx_v[pl.ds(c * _LANES, _LANES)] = (
                base + c * _LANES + lax.iota(jnp.int32, _LANES)
            )
        pltpu.async_copy(table_hbm.at[idx_v], rows_v, sem).wait()
        pltpu.sync_copy(rows_v, out_hbm.at[pl.ds(base, rows_per_w)])

    return gather_k(pos_embed)


def _tc_body(pe_ref, out_ref):
    pe = pe_ref[...]
    out_ref[...] = jnp.broadcast_to(pe[:, None, :], out_ref.shape)


def kernel(z, pos_embed):
    batch = z.shape[1]
    rows = _sc_gather(pos_embed)
    out = pl.pallas_call(
        _tc_body,
        grid=(_SEQ // _S_BLK,),
        in_specs=[pl.BlockSpec((_S_BLK, _D), lambda i: (i, 0))],
        out_specs=pl.BlockSpec((_S_BLK, batch, _D), lambda i: (i, 0, 0)),
        out_shape=jax.ShapeDtypeStruct((_SEQ, batch, _D), z.dtype),
    )(rows)
    return out


# speedup vs baseline: 1.2755x; 1.0102x over previous
import functools
import jax
import jax.numpy as jnp
from jax import lax
from jax.experimental import pallas as pl
from jax.experimental.pallas import tpu as pltpu
from jax.experimental.pallas import tpu_sc as plsc

_SEQ = 2048
_D = 256
_NC = 2
_NS = 16
_LANES = 16
_S_BLK = 64


def _sc_gather(pos_embed):
    nw = _NC * _NS
    rows_per_w = _SEQ // nw

    mesh = plsc.VectorSubcoreMesh(core_axis_name="c", subcore_axis_name="s")

    half = rows_per_w // 2

    @functools.partial(
        pl.kernel,
        mesh=mesh,
        out_type=jax.ShapeDtypeStruct((_SEQ, _D), jnp.float32),
        scratch_types=[
            pltpu.VMEM((rows_per_w,), jnp.int32),
            pltpu.VMEM((half, _D), jnp.float32),
            pltpu.VMEM((half, _D), jnp.float32),
            pltpu.SemaphoreType.DMA,
            pltpu.SemaphoreType.DMA,
            pltpu.SemaphoreType.DMA,
            pltpu.SemaphoreType.DMA,
        ],
    )
    def gather_k(table_hbm, out_hbm, idx_v, rows0_v, rows1_v, g0, g1, o0, o1):
        wid = lax.axis_index("s") * _NC + lax.axis_index("c")
        base = wid * rows_per_w
        for c in range(rows_per_w // _LANES):
            idx_v[pl.ds(c * _LANES, _LANES)] = (
                base + c * _LANES + lax.iota(jnp.int32, _LANES)
            )
        h0 = pltpu.async_copy(table_hbm.at[idx_v.at[pl.ds(0, half)]], rows0_v, g0)
        h1 = pltpu.async_copy(table_hbm.at[idx_v.at[pl.ds(half, half)]], rows1_v, g1)
        h0.wait()
        w0 = pltpu.async_copy(rows0_v, out_hbm.at[pl.ds(base, half)], o0)
        h1.wait()
        w1 = pltpu.async_copy(rows1_v, out_hbm.at[pl.ds(base + half, half)], o1)
        w0.wait()
        w1.wait()

    return gather_k(pos_embed)


def _tc_body(pe_ref, out_ref):
    pe = pe_ref[...]
    out_ref[...] = jnp.broadcast_to(pe[:, None, :], out_ref.shape)


def kernel(z, pos_embed):
    batch = z.shape[1]
    rows = _sc_gather(pos_embed)
    out = pl.pallas_call(
        _tc_body,
        grid=(_SEQ // _S_BLK,),
        in_specs=[pl.BlockSpec((_S_BLK, _D), lambda i: (i, 0))],
        out_specs=pl.BlockSpec((_S_BLK, batch, _D), lambda i: (i, 0, 0)),
        out_shape=jax.ShapeDtypeStruct((_SEQ, batch, _D), z.dtype),
    )(rows)
    return out
